# Initial kernel scaffold; baseline (speedup 1.0000x reference)
import jax
import jax.numpy as jnp
from jax.experimental import pallas as pl


def kernel(x, wx0, wh0, b0, wx1, wh1, b1):
    raise NotImplementedError("write your pallas kernel here")


# speedup vs baseline: 2.4299x; 2.4299x over previous
import functools

import jax
import jax.numpy as jnp
from jax.experimental import pallas as pl
from jax.experimental.pallas import tpu as pltpu


def _round_up(x, m):
    return ((x + m - 1) // m) * m


def _fused_convrnn_kernel(x_ref, w_ref, b_ref, m_ref, y_ref,
                          h1_ref, h2_ref, slab_ref, *,
                          T, cin, hd, kh, kw, Wp, NP, OFF, splits):
    ph, pw = kh // 2, kw // 2
    offs = [OFF + (ki - ph) * Wp + (kj - pw)
            for ki in range(kh) for kj in range(kw)]
    KX = kh * kw * cin

    h1_ref[...] = jnp.zeros_like(h1_ref)
    h2_ref[...] = jnp.zeros_like(h2_ref)

    for k in range(T + 1):
        if k < T:
            for tap, o in enumerate(offs):
                slab_ref[tap * cin:(tap + 1) * cin, :] = x_ref[k, :, o:o + NP]
        for tap, o in enumerate(offs):
            r = KX + tap * hd
            slab_ref[r:r + hd, :] = h1_ref[:, o:o + NP]
        for tap, o in enumerate(offs):
            r = KX + (kh * kw + tap) * hd
            slab_ref[r:r + hd, :] = h2_ref[:, o:o + NP]
        for s, nw in splits:
            acc = jnp.dot(w_ref[...], slab_ref[:, s:s + nw],
                          preferred_element_type=jnp.float32)
            g = jnp.tanh(acc + b_ref[...]) * m_ref[:, s:s + nw]
            if k < T:
                h1_ref[:, OFF + s:OFF + s + nw] = g[:hd].astype(h1_ref.dtype)
            if k >= 1:
                y_ref[k - 1, :, s:s + nw] = g[hd:]
                h2_ref[:, OFF + s:OFF + s + nw] = g[hd:].astype(h2_ref.dtype)


def _gate_slices(wx, wh, b, hd):
    wxg = wx[..., 3 * hd:4 * hd]
    whg = wh[..., 3 * hd:4 * hd]
    bg = b[:, 3 * hd:4 * hd].reshape(hd)
    wx2 = wxg.transpose(3, 0, 1, 2).reshape(hd, -1)
    wh2 = whg.transpose(3, 0, 1, 2).reshape(hd, -1)
    return wx2, wh2, bg


def kernel(x, wx0, wh0, b0, wx1, wh1, b1):
    T, B, cin, H, W = x.shape
    hd = wx0.shape[-1] // 4
    kh, kw = wx0.shape[0], wx0.shape[1]
    ph, pw = kh // 2, kw // 2
    Hp, Wp = H + 2 * ph, W + 2 * pw
    NP = _round_up(Hp * Wp, 128)
    OFF = _round_up(max(ph * Wp + pw, 1), 128)
    EXT = OFF + NP + OFF
    KX, KH = kh * kw * cin, kh * kw * hd
    K = KX + 2 * KH

    splits = (((0, NP // 2), (NP // 2, NP // 2)) if NP % 256 == 0
              else ((0, NP),))

    wx2_0, wh2_0, bg0 = _gate_slices(wx0, wh0, b0, hd)
    wx2_1, wh2_1, bg1 = _gate_slices(wx1, wh1, b1, hd)
    z_xh = jnp.zeros((hd, KX), jnp.float32)
    z_hh = jnp.zeros((hd, KH), jnp.float32)
    w_top = jnp.concatenate([wx2_0, wh2_0, z_hh], axis=1)
    w_bot = jnp.concatenate([z_xh, wx2_1, wh2_1], axis=1)
    w = jnp.concatenate([w_top, w_bot], axis=0).astype(jnp.bfloat16)
    bias = jnp.concatenate([bg0, bg1]).reshape(2 * hd, 1)

    xb = x.transpose(1, 0, 2, 3, 4).astype(jnp.float32)
    xb = jnp.pad(xb, ((0, 0), (0, 0), (0, 0), (ph, ph), (pw, pw)))
    xb = xb.reshape(B, T, cin, Hp * Wp)
    xb = jnp.pad(xb, ((0, 0), (0, 0), (0, 0), (OFF, EXT - OFF - Hp * Wp)))
    xb = xb.astype(jnp.bfloat16)

    idx = jnp.arange(NP)
    r, c = idx // Wp, idx % Wp
    m = ((r >= ph) & (r < Hp - ph) & (c >= pw) & (c < Wp - pw)
         & (idx < Hp * Wp))
    mask = m.astype(jnp.float32).reshape(1, NP)

    body = functools.partial(_fused_convrnn_kernel, T=T, cin=cin, hd=hd,
                             kh=kh, kw=kw, Wp=Wp, NP=NP, OFF=OFF,
                             splits=splits)

    y = pl.pallas_call(
        body,
        out_shape=jax.ShapeDtypeStruct((B, T, hd, NP), jnp.float32),
        grid=(B,),
        in_specs=[
            pl.BlockSpec((None, T, cin, EXT), lambda b: (b, 0, 0, 0)),
            pl.BlockSpec((2 * hd, K), lambda b: (0, 0)),
            pl.BlockSpec((2 * hd, 1), lambda b: (0, 0)),
            pl.BlockSpec((1, NP), lambda b: (0, 0)),
        ],
        out_specs=pl.BlockSpec((None, T, hd, NP), lambda b: (b, 0, 0, 0)),
        scratch_shapes=[
            pltpu.VMEM((hd, EXT), jnp.bfloat16),
            pltpu.VMEM((hd, EXT), jnp.bfloat16),
            pltpu.VMEM((K, NP), jnp.bfloat16),
        ],
        compiler_params=pltpu.CompilerParams(
            dimension_semantics=("arbitrary",),
        ),
        name="fused_convrnn2",
    )(xb, w, bias, mask)

    y = y[..., :Hp * Wp].reshape(B, T, hd, Hp, Wp)
    return y[:, :, :, ph:ph + H, pw:pw + W]
